# Initial kernel scaffold; baseline (speedup 1.0000x reference)
import jax
import jax.numpy as jnp
from jax.experimental import pallas as pl


def kernel(h_t, h_q, mm, matching_matrix, mask, target_edge_index, target_batch, query_edge_index, query_batch, params, ly):
    raise NotImplementedError("write your pallas kernel here")


# speedup vs baseline: 5.3630x; 5.3630x over previous
import functools

import jax
import jax.numpy as jnp
from jax.experimental import pallas as pl
from jax.experimental.pallas import tpu as pltpu

H = 128
HEADS = 8
NB = 16

N_T = 10000
N_Q = 2048


def _elu(x):
    return jnp.where(x > 0, x, jnp.exp(x) - 1.0)


def _k1_body(mmat_ref, ht_ref, thr_ref, htg_ref):
    mx = jnp.max(mmat_ref[...], axis=0, keepdims=True)
    cb = mx.shape[1]
    rows = jax.lax.broadcasted_iota(jnp.int32, (cb, cb), 0)
    cols = jax.lax.broadcasted_iota(jnp.int32, (cb, cb), 1)
    ident = (rows == cols).astype(jnp.float32)
    gate_row = (mx > thr_ref[0, 0]).astype(jnp.float32)
    gate_col = jax.lax.dot_general(
        ident, gate_row, (((1,), (1,)), ((), ())),
        preferred_element_type=jnp.float32)
    htg_ref[...] = ht_ref[...] * gate_col


def _gate_ht(matching_matrix, h_t, thr):
    cb = 1280
    grid = (pl.cdiv(N_T, cb),)
    return pl.pallas_call(
        _k1_body,
        grid=grid,
        in_specs=[
            pl.BlockSpec((N_Q, cb), lambda j: (0, j)),
            pl.BlockSpec((cb, H), lambda j: (j, 0)),
            pl.BlockSpec((1, 1), lambda j: (0, 0)),
        ],
        out_specs=pl.BlockSpec((cb, H), lambda j: (j, 0)),
        out_shape=jax.ShapeDtypeStruct((N_T, H), jnp.float32),
    )(matching_matrix, h_t, thr)


def _mm_body(a_ref, b_ref, o_ref):
    k = pl.program_id(1)

    @pl.when(k == 0)
    def _():
        o_ref[...] = jnp.zeros_like(o_ref)

    o_ref[...] += jnp.dot(a_ref[...], b_ref[...],
                          preferred_element_type=jnp.float32)


def _matmul(a, b, bm, bk):
    m, k = a.shape
    _, n = b.shape
    return pl.pallas_call(
        _mm_body,
        grid=(m // bm, k // bk),
        in_specs=[
            pl.BlockSpec((bm, bk), lambda i, j: (i, j)),
            pl.BlockSpec((bk, n), lambda i, j: (j, 0)),
        ],
        out_specs=pl.BlockSpec((bm, n), lambda i, j: (i, 0)),
        out_shape=jax.ShapeDtypeStruct((m, n), jnp.float32),
        compiler_params=pltpu.CompilerParams(
            dimension_semantics=("parallel", "arbitrary")),
    )(a, b)


def _pool_body(hq_ref, qb_ref, gw_ref, gb_ref,
               w1_ref, b1_ref, g1_ref, be1_ref,
               w2_ref, b2_ref, g2_ref, be2_ref, out_ref):
    hq = hq_ref[...]
    logits = jnp.dot(hq, gw_ref[...],
                     preferred_element_type=jnp.float32) + gb_ref[0, 0]
    qb = qb_ref[...]
    lanes = jax.lax.broadcasted_iota(jnp.int32, (N_Q, NB), 1)
    onehot = qb == lanes
    x = jnp.where(onehot, logits, -1e30)
    m = jnp.max(x, axis=0, keepdims=True)
    e = jnp.exp(x - m) * onehot.astype(jnp.float32)
    s = jnp.sum(e, axis=0, keepdims=True)
    w = e / (s + 1e-16)
    q = jax.lax.dot_general(w, hq, (((0,), (0,)), ((), ())),
                            preferred_element_type=jnp.float32)
    y = jnp.dot(q, w1_ref[...], preferred_element_type=jnp.float32) + b1_ref[...]
    mn = jnp.mean(y, axis=0, keepdims=True)
    vr = jnp.mean((y - mn) ** 2, axis=0, keepdims=True)
    y = _elu((y - mn) / jnp.sqrt(vr + 1e-5) * g1_ref[...] + be1_ref[...])
    y = jnp.dot(y, w2_ref[...], preferred_element_type=jnp.float32) + b2_ref[...]
    mn2 = jnp.mean(y, axis=0, keepdims=True)
    vr2 = jnp.mean((y - mn2) ** 2, axis=0, keepdims=True)
    out_ref[...] = _elu((y - mn2) / jnp.sqrt(vr2 + 1e-5) * g2_ref[...] + be2_ref[...])


def _pool_mlp0(h_q, qb_col, p):
    m = p['mlp0']
    full = lambda shp: pl.BlockSpec(shp, lambda: (0, 0))
    args = [h_q, qb_col, p['gate_W'], p['gate_b'].reshape(1, 1),
            m['W1'], m['b1'].reshape(1, -1), m['g1'].reshape(1, -1),
            m['be1'].reshape(1, -1),
            m['W2'], m['b2'].reshape(1, -1), m['g2'].reshape(1, -1),
            m['be2'].reshape(1, -1)]
    return pl.pallas_call(
        _pool_body,
        in_specs=[full(a.shape) for a in args],
        out_specs=full((NB, 2 * HEADS * H)),
        out_shape=jax.ShapeDtypeStruct((NB, 2 * HEADS * H), jnp.float32),
    )(*args)


def _k3_body(htg_ref, tb_ref, a0_ref, gatw_ref, z_ref):
    htg = htg_ref[...]
    rb = htg.shape[0]
    xlin = jnp.dot(htg, gatw_ref[...],
                   preferred_element_type=jnp.float32)
    tb = tb_ref[...]
    onehot = (tb == jax.lax.broadcasted_iota(jnp.int32, (rb, NB), 1)
              ).astype(jnp.float32)
    att0 = jnp.dot(onehot, a0_ref[...],
                   preferred_element_type=jnp.float32)
    prod = xlin * att0
    zs = []
    for h in range(HEADS):
        sl = slice(h * H, (h + 1) * H)
        alpha = jnp.sum(prod[:, sl], axis=1, keepdims=True)
        zs.append(xlin[:, sl] * jax.nn.sigmoid(alpha))
    z_ref[...] = jnp.concatenate(zs, axis=1)


def _gat_messages(htg, tb_col, a0, gat_W):
    rb = 1000
    grid = (N_T // rb,)
    return pl.pallas_call(
        _k3_body,
        grid=grid,
        in_specs=[
            pl.BlockSpec((rb, H), lambda i: (i, 0)),
            pl.BlockSpec((rb, 1), lambda i: (i, 0)),
            pl.BlockSpec((NB, HEADS * H), lambda i: (0, 0)),
            pl.BlockSpec((H, HEADS * H), lambda i: (0, 0)),
        ],
        out_specs=pl.BlockSpec((rb, HEADS * H), lambda i: (i, 0)),
        out_shape=jax.ShapeDtypeStruct((N_T, HEADS * H), jnp.float32),
    )(htg, tb_col, a0, gat_W)


def _k4a_body(x_ref, w1_ref, b1_ref, y1_ref, s1_ref, q1_ref):
    i = pl.program_id(0)

    @pl.when(i == 0)
    def _():
        s1_ref[...] = jnp.zeros_like(s1_ref)
        q1_ref[...] = jnp.zeros_like(q1_ref)

    y = jnp.dot(x_ref[...], w1_ref[...],
                preferred_element_type=jnp.float32) + b1_ref[...]
    y1_ref[...] = y
    s1_ref[...] += jnp.sum(y, axis=0, keepdims=True)
    q1_ref[...] += jnp.sum(y * y, axis=0, keepdims=True)


def _k4b_body(y1_ref, s1_ref, q1_ref, g1_ref, be1_ref, w2_ref, b2_ref,
              y2_ref, s2_ref, q2_ref):
    i = pl.program_id(0)

    @pl.when(i == 0)
    def _():
        s2_ref[...] = jnp.zeros_like(s2_ref)
        q2_ref[...] = jnp.zeros_like(q2_ref)

    n = jnp.float32(N_T)
    mn = s1_ref[...] / n
    vr = q1_ref[...] / n - mn * mn
    a = _elu((y1_ref[...] - mn) / jnp.sqrt(vr + 1e-5) * g1_ref[...] + be1_ref[...])
    y = jnp.dot(a, w2_ref[...], preferred_element_type=jnp.float32) + b2_ref[...]
    y2_ref[...] = y
    s2_ref[...] += jnp.sum(y, axis=0, keepdims=True)
    q2_ref[...] += jnp.sum(y * y, axis=0, keepdims=True)


def _k4c_body(y2_ref, s2_ref, q2_ref, g2_ref, be2_ref, htg_ref, out_ref):
    n = jnp.float32(N_T)
    mn = s2_ref[...] / n
    vr = q2_ref[...] / n - mn * mn
    mlp = _elu((y2_ref[...] - mn) / jnp.sqrt(vr + 1e-5) * g2_ref[...] + be2_ref[...])
    htg = htg_ref[...]
    out_ref[...] = jnp.where(htg != 0, mlp + htg, htg)


def _mlp1_residual(x, htg, p):
    m = p['mlp1']
    rb = 1000
    grid = (N_T // rb,)
    dh = m['W1'].shape[1]
    arb = pltpu.CompilerParams(dimension_semantics=("arbitrary",))
    y1, s1, q1 = pl.pallas_call(
        _k4a_body,
        grid=grid,
        in_specs=[
            pl.BlockSpec((rb, HEADS * H), lambda i: (i, 0)),
            pl.BlockSpec((HEADS * H, dh), lambda i: (0, 0)),
            pl.BlockSpec((1, dh), lambda i: (0, 0)),
        ],
        out_specs=[
            pl.BlockSpec((rb, dh), lambda i: (i, 0)),
            pl.BlockSpec((1, dh), lambda i: (0, 0)),
            pl.BlockSpec((1, dh), lambda i: (0, 0)),
        ],
        out_shape=[
            jax.ShapeDtypeStruct((N_T, dh), jnp.float32),
            jax.ShapeDtypeStruct((1, dh), jnp.float32),
            jax.ShapeDtypeStruct((1, dh), jnp.float32),
        ],
        compiler_params=arb,
    )(x, m['W1'], m['b1'].reshape(1, -1))
    y2, s2, q2 = pl.pallas_call(
        _k4b_body,
        grid=grid,
        in_specs=[
            pl.BlockSpec((rb, dh), lambda i: (i, 0)),
            pl.BlockSpec((1, dh), lambda i: (0, 0)),
            pl.BlockSpec((1, dh), lambda i: (0, 0)),
            pl.BlockSpec((1, dh), lambda i: (0, 0)),
            pl.BlockSpec((1, dh), lambda i: (0, 0)),
            pl.BlockSpec((dh, H), lambda i: (0, 0)),
            pl.BlockSpec((1, H), lambda i: (0, 0)),
        ],
        out_specs=[
            pl.BlockSpec((rb, H), lambda i: (i, 0)),
            pl.BlockSpec((1, H), lambda i: (0, 0)),
            pl.BlockSpec((1, H), lambda i: (0, 0)),
        ],
        out_shape=[
            jax.ShapeDtypeStruct((N_T, H), jnp.float32),
            jax.ShapeDtypeStruct((1, H), jnp.float32),
            jax.ShapeDtypeStruct((1, H), jnp.float32),
        ],
        compiler_params=arb,
    )(y1, s1, q1, m['g1'].reshape(1, -1), m['be1'].reshape(1, -1),
      m['W2'], m['b2'].reshape(1, -1))
    return pl.pallas_call(
        _k4c_body,
        grid=grid,
        in_specs=[
            pl.BlockSpec((rb, H), lambda i: (i, 0)),
            pl.BlockSpec((1, H), lambda i: (0, 0)),
            pl.BlockSpec((1, H), lambda i: (0, 0)),
            pl.BlockSpec((1, H), lambda i: (0, 0)),
            pl.BlockSpec((1, H), lambda i: (0, 0)),
            pl.BlockSpec((rb, H), lambda i: (i, 0)),
        ],
        out_specs=pl.BlockSpec((rb, H), lambda i: (i, 0)),
        out_shape=jax.ShapeDtypeStruct((N_T, H), jnp.float32),
    )(y2, s2, q2, m['g2'].reshape(1, -1), m['be2'].reshape(1, -1), htg)


def _k6_body(xa_ref, xb_ref, bias_ref, hq_ref,
             w1_ref, b1_ref, g1_ref, be1_ref,
             w2_ref, b2_ref, g2_ref, be2_ref, out_ref):
    x = xa_ref[...] + xb_ref[...] + bias_ref[...]
    y = jnp.dot(x, w1_ref[...], preferred_element_type=jnp.float32) + b1_ref[...]
    mn = jnp.mean(y, axis=0, keepdims=True)
    vr = jnp.mean((y - mn) ** 2, axis=0, keepdims=True)
    y = _elu((y - mn) / jnp.sqrt(vr + 1e-5) * g1_ref[...] + be1_ref[...])
    y = jnp.dot(y, w2_ref[...], preferred_element_type=jnp.float32) + b2_ref[...]
    mn2 = jnp.mean(y, axis=0, keepdims=True)
    vr2 = jnp.mean((y - mn2) ** 2, axis=0, keepdims=True)
    y = _elu((y - mn2) / jnp.sqrt(vr2 + 1e-5) * g2_ref[...] + be2_ref[...])
    out_ref[...] = y + hq_ref[...]


def _mlp2_residual(xa, xb, bias, h_q, p):
    m = p['mlp2']
    full = lambda shp: pl.BlockSpec(shp, lambda: (0, 0))
    args = [xa, xb, bias, h_q,
            m['W1'], m['b1'].reshape(1, -1), m['g1'].reshape(1, -1),
            m['be1'].reshape(1, -1),
            m['W2'], m['b2'].reshape(1, -1), m['g2'].reshape(1, -1),
            m['be2'].reshape(1, -1)]
    return pl.pallas_call(
        _k6_body,
        in_specs=[full(a.shape) for a in args],
        out_specs=full((N_Q, H)),
        out_shape=jax.ShapeDtypeStruct((N_Q, H), jnp.float32),
    )(*args)


def _k5_body(hq_ref, ht_ref, mask_ref, stau_ref, mm_ref, mmat_ref):
    hq = hq_ref[...]
    ht = ht_ref[...]
    dot = jax.lax.dot_general(hq, ht, (((1,), (1,)), ((), ())),
                              preferred_element_type=jnp.float32)
    nq = jnp.sqrt(jnp.sum(hq * hq, axis=1, keepdims=True))
    ones = jnp.ones((1, H), jnp.float32)
    nt2 = jax.lax.dot_general(ones, ht * ht, (((1,), (1,)), ((), ())),
                              preferred_element_type=jnp.float32)
    nrm = nq * jnp.sqrt(nt2)
    cosm = jnp.where(nrm != 0, dot / (nrm + 1e-9), -1.0)
    stau = stau_ref[0, 0]
    maskf = mask_ref[...].astype(jnp.float32)
    mmat = cosm * maskf / stau + (-1e9) * (1.0 - maskf)
    mx = jnp.max(mmat, axis=1, keepdims=True)
    e = jnp.exp(mmat - mx)
    s = jnp.sum(e, axis=1, keepdims=True)
    mm_ref[...] = e / s
    mmat_ref[...] = mmat * stau


def _matching(h_q_new, h_t_new, mask_i8, stau):
    rb = 128
    grid = (N_Q // rb,)
    return pl.pallas_call(
        _k5_body,
        grid=grid,
        in_specs=[
            pl.BlockSpec((rb, H), lambda i: (i, 0)),
            pl.BlockSpec((N_T, H), lambda i: (0, 0)),
            pl.BlockSpec((rb, N_T), lambda i: (i, 0)),
            pl.BlockSpec((1, 1), lambda i: (0, 0)),
        ],
        out_specs=[
            pl.BlockSpec((rb, N_T), lambda i: (i, 0)),
            pl.BlockSpec((rb, N_T), lambda i: (i, 0)),
        ],
        out_shape=[
            jax.ShapeDtypeStruct((N_Q, N_T), jnp.float32),
            jax.ShapeDtypeStruct((N_Q, N_T), jnp.float32),
        ],
    )(h_q_new, h_t_new, mask_i8, stau)


def kernel(h_t, h_q, mm, matching_matrix, mask, target_edge_index,
           target_batch, query_edge_index, query_batch, params, ly):
    p = params
    thr = jnp.tanh(p['beta']).reshape(1, 1)
    stau = jax.nn.sigmoid(p['tau']).reshape(1, 1)

    htg = _gate_ht(matching_matrix, h_t, thr)
    n = _matmul(mm, htg, 256, N_T)

    q_mlp = _pool_mlp0(h_q, query_batch.reshape(N_Q, 1).astype(jnp.int32), p)
    a0 = q_mlp.reshape(NB, HEADS, 2 * H)[:, :, :H].reshape(NB, HEADS * H)

    z = _gat_messages(htg, target_batch.reshape(N_T, 1).astype(jnp.int32),
                      a0, p['gat_W'])

    src, dst = target_edge_index[0], target_edge_index[1]
    h_t_gat = jax.ops.segment_sum(z[src], dst, num_segments=N_T)

    xw = _matmul(n, p['gcn_W'], 256, 128)
    qs, qd = query_edge_index[0], query_edge_index[1]
    loop = jnp.arange(N_Q, dtype=qs.dtype)
    s_all = jnp.concatenate([qs, loop])
    d_all = jnp.concatenate([qd, loop])
    deg = jax.ops.segment_sum(jnp.ones_like(s_all, dtype=jnp.float32),
                              d_all, num_segments=N_Q)
    dinv = jnp.where(deg > 0, deg ** -0.5, 0.0)
    norm = dinv[s_all] * dinv[d_all]
    h_q_gat = jax.ops.segment_sum(norm[:, None] * xw[s_all], d_all,
                                  num_segments=N_Q)

    h_t_new = _mlp1_residual(h_t_gat, htg, p)
    h_q_new = _mlp2_residual(h_q_gat, jnp.zeros_like(h_q_gat),
                             jnp.broadcast_to(p['gcn_b'].reshape(1, H), (N_Q, H)),
                             h_q, p)

    mm_out, mmat_stau = _matching(h_q_new, h_t_new,
                                  mask.astype(jnp.int8), stau)
    return (h_t_new, h_q_new, mm_out, mmat_stau)


# speedup vs baseline: 20.5095x; 3.8243x over previous
import functools

import jax
import jax.numpy as jnp
from jax import lax
from jax.experimental import pallas as pl
from jax.experimental.pallas import tpu as pltpu
from jax.experimental.pallas import tpu_sc as plsc

NC = 2
NS = 16

H = 128
HEADS = 8
NB = 16

N_T = 10000
N_Q = 2048


def _elu(x):
    return jnp.where(x > 0, x, jnp.exp(x) - 1.0)


def _k1_body(mmat_ref, ht_ref, thr_ref, htg_ref):
    mx = jnp.max(mmat_ref[...], axis=0, keepdims=True)
    cb = mx.shape[1]
    rows = jax.lax.broadcasted_iota(jnp.int32, (cb, cb), 0)
    cols = jax.lax.broadcasted_iota(jnp.int32, (cb, cb), 1)
    ident = (rows == cols).astype(jnp.float32)
    gate_row = (mx > thr_ref[0, 0]).astype(jnp.float32)
    gate_col = jax.lax.dot_general(
        ident, gate_row, (((1,), (1,)), ((), ())),
        preferred_element_type=jnp.float32)
    htg_ref[...] = ht_ref[...] * gate_col


def _gate_ht(matching_matrix, h_t, thr):
    cb = 1280
    grid = (pl.cdiv(N_T, cb),)
    return pl.pallas_call(
        _k1_body,
        grid=grid,
        in_specs=[
            pl.BlockSpec((N_Q, cb), lambda j: (0, j)),
            pl.BlockSpec((cb, H), lambda j: (j, 0)),
            pl.BlockSpec((1, 1), lambda j: (0, 0)),
        ],
        out_specs=pl.BlockSpec((cb, H), lambda j: (j, 0)),
        out_shape=jax.ShapeDtypeStruct((N_T, H), jnp.float32),
    )(matching_matrix, h_t, thr)


def _mm_body(a_ref, b_ref, o_ref):
    k = pl.program_id(1)

    @pl.when(k == 0)
    def _():
        o_ref[...] = jnp.zeros_like(o_ref)

    o_ref[...] += jnp.dot(a_ref[...], b_ref[...],
                          preferred_element_type=jnp.float32)


def _matmul(a, b, bm, bk):
    m, k = a.shape
    _, n = b.shape
    return pl.pallas_call(
        _mm_body,
        grid=(m // bm, k // bk),
        in_specs=[
            pl.BlockSpec((bm, bk), lambda i, j: (i, j)),
            pl.BlockSpec((bk, n), lambda i, j: (j, 0)),
        ],
        out_specs=pl.BlockSpec((bm, n), lambda i, j: (i, 0)),
        out_shape=jax.ShapeDtypeStruct((m, n), jnp.float32),
        compiler_params=pltpu.CompilerParams(
            dimension_semantics=("parallel", "arbitrary")),
    )(a, b)


def _pool_body(hq_ref, qb_ref, gw_ref, gb_ref,
               w1_ref, b1_ref, g1_ref, be1_ref,
               w2_ref, b2_ref, g2_ref, be2_ref, out_ref):
    hq = hq_ref[...]
    logits = jnp.dot(hq, gw_ref[...],
                     preferred_element_type=jnp.float32) + gb_ref[0, 0]
    qb = qb_ref[...]
    lanes = jax.lax.broadcasted_iota(jnp.int32, (N_Q, NB), 1)
    onehot = qb == lanes
    x = jnp.where(onehot, logits, -1e30)
    m = jnp.max(x, axis=0, keepdims=True)
    e = jnp.exp(x - m) * onehot.astype(jnp.float32)
    s = jnp.sum(e, axis=0, keepdims=True)
    w = e / (s + 1e-16)
    q = jax.lax.dot_general(w, hq, (((0,), (0,)), ((), ())),
                            preferred_element_type=jnp.float32)
    y = jnp.dot(q, w1_ref[...], preferred_element_type=jnp.float32) + b1_ref[...]
    mn = jnp.mean(y, axis=0, keepdims=True)
    vr = jnp.mean((y - mn) ** 2, axis=0, keepdims=True)
    y = _elu((y - mn) / jnp.sqrt(vr + 1e-5) * g1_ref[...] + be1_ref[...])
    y = jnp.dot(y, w2_ref[...], preferred_element_type=jnp.float32) + b2_ref[...]
    mn2 = jnp.mean(y, axis=0, keepdims=True)
    vr2 = jnp.mean((y - mn2) ** 2, axis=0, keepdims=True)
    out_ref[...] = _elu((y - mn2) / jnp.sqrt(vr2 + 1e-5) * g2_ref[...] + be2_ref[...])


def _pool_mlp0(h_q, qb_col, p):
    m = p['mlp0']
    full = lambda shp: pl.BlockSpec(shp, lambda: (0, 0))
    args = [h_q, qb_col, p['gate_W'], p['gate_b'].reshape(1, 1),
            m['W1'], m['b1'].reshape(1, -1), m['g1'].reshape(1, -1),
            m['be1'].reshape(1, -1),
            m['W2'], m['b2'].reshape(1, -1), m['g2'].reshape(1, -1),
            m['be2'].reshape(1, -1)]
    return pl.pallas_call(
        _pool_body,
        in_specs=[full(a.shape) for a in args],
        out_specs=full((NB, 2 * HEADS * H)),
        out_shape=jax.ShapeDtypeStruct((NB, 2 * HEADS * H), jnp.float32),
    )(*args)


def _k3_body(htg_ref, tb_ref, a0_ref, gatw_ref, z_ref):
    htg = htg_ref[...]
    rb = htg.shape[0]
    xlin = jnp.dot(htg, gatw_ref[...],
                   preferred_element_type=jnp.float32)
    tb = tb_ref[...]
    onehot = (tb == jax.lax.broadcasted_iota(jnp.int32, (rb, NB), 1)
              ).astype(jnp.float32)
    att0 = jnp.dot(onehot, a0_ref[...],
                   preferred_element_type=jnp.float32)
    prod = xlin * att0
    for h in range(HEADS):
        sl = slice(h * H, (h + 1) * H)
        alpha = jnp.sum(prod[:, sl], axis=1, keepdims=True)
        z_ref[h] = xlin[:, sl] * jax.nn.sigmoid(alpha)


def _gat_messages(htg, tb_col, a0, gat_W):
    rb = 1000
    grid = (N_T // rb,)
    return pl.pallas_call(
        _k3_body,
        grid=grid,
        in_specs=[
            pl.BlockSpec((rb, H), lambda i: (i, 0)),
            pl.BlockSpec((rb, 1), lambda i: (i, 0)),
            pl.BlockSpec((NB, HEADS * H), lambda i: (0, 0)),
            pl.BlockSpec((H, HEADS * H), lambda i: (0, 0)),
        ],
        out_specs=pl.BlockSpec((HEADS, rb, H), lambda i: (0, i, 0)),
        out_shape=jax.ShapeDtypeStruct((HEADS, N_T, H), jnp.float32),
    )(htg, tb_col, a0, gat_W)


_STRIPE_T = 624
_TAIL_T = N_T - NS * _STRIPE_T


def _gat_sc(z3, src, dst, zeros_stripe):
    et = src.shape[0]
    per_tile = et // NS
    k = 80
    nchunks = per_tile // k
    hpc = HEADS // NC
    mesh = plsc.VectorSubcoreMesh(core_axis_name="c", subcore_axis_name="s")

    @functools.partial(
        pl.kernel, mesh=mesh,
        out_type=jax.ShapeDtypeStruct((HEADS, N_T, H), jnp.float32),
        scratch_types=[
            pltpu.VMEM((2, k), jnp.int32),
            pltpu.VMEM((2, k), jnp.int32),
            pltpu.VMEM((2, k, H), jnp.float32),
            pltpu.VMEM_SHARED((N_T, H), jnp.float32),
            pltpu.SemaphoreType.DMA,
            pltpu.SemaphoreType.DMA,
        ],
    )
    def kern(z3_hbm, src_hbm, dst_hbm, zro_hbm, out_hbm,
             sidx, didx, rows, acc, gsem0, gsem1):
        c = lax.axis_index("c")
        s = lax.axis_index("s")
        base = s * per_tile
        gsems = (gsem0, gsem1)

        for h in range(hpc):
            head = c * hpc + h

            pltpu.sync_copy(zro_hbm, acc.at[pl.ds(s * _STRIPE_T, _STRIPE_T)])

            @pl.when(s == NS - 1)
            def _():
                pltpu.sync_copy(zro_hbm.at[pl.ds(0, _TAIL_T)],
                                acc.at[pl.ds(NS * _STRIPE_T, _TAIL_T)])

            plsc.subcore_barrier()

            def _fetch(b, j):
                off = base + j * k
                pltpu.sync_copy(src_hbm.at[pl.ds(off, k)], sidx.at[b])
                pltpu.sync_copy(dst_hbm.at[pl.ds(off, k)], didx.at[b])
                pltpu.async_copy(z3_hbm.at[head].at[sidx.at[b]],
                                 rows.at[b], gsems[b])

            _fetch(0, 0)
            _fetch(1, 1)

            def _step(i, carry):
                for b in range(2):
                    j = 2 * i + b
                    pltpu.make_async_copy(z3_hbm.at[head].at[sidx.at[b]],
                                          rows.at[b], gsems[b]).wait()
                    pltpu.sync_copy(rows.at[b], acc.at[didx.at[b]], add=True)

                    @pl.when(j + 2 < nchunks)
                    def _():
                        _fetch(b, j + 2)
                return carry

            lax.fori_loop(0, nchunks // 2, _step, 0)
            plsc.subcore_barrier()
            pltpu.sync_copy(acc.at[pl.ds(s * _STRIPE_T, _STRIPE_T)],
                            out_hbm.at[head, pl.ds(s * _STRIPE_T, _STRIPE_T)])

            @pl.when(s == NS - 1)
            def _():
                pltpu.sync_copy(acc.at[pl.ds(NS * _STRIPE_T, _TAIL_T)],
                                out_hbm.at[head, pl.ds(NS * _STRIPE_T, _TAIL_T)])

    return kern(z3, src, dst, zeros_stripe)


def _gcn_sc(xwn, qs, qd, zeros_stripe):
    et = qs.shape[0]
    per_tile = et // (NC * NS)
    k = 128
    nchunks = per_tile // k
    stripe = N_Q // NS
    mesh = plsc.VectorSubcoreMesh(core_axis_name="c", subcore_axis_name="s")

    @functools.partial(
        pl.kernel, mesh=mesh,
        out_type=jax.ShapeDtypeStruct((NC, N_Q, H), jnp.float32),
        scratch_types=[
            pltpu.VMEM((2, k), jnp.int32),
            pltpu.VMEM((2, k), jnp.int32),
            pltpu.VMEM((2, k, H), jnp.float32),
            pltpu.VMEM_SHARED((N_Q, H), jnp.float32),
            pltpu.SemaphoreType.DMA,
            pltpu.SemaphoreType.DMA,
        ],
    )
    def kern(xwn_hbm, qs_hbm, qd_hbm, zro_hbm, out_hbm,
             sidx, didx, rows, acc, gsem0, gsem1):
        c = lax.axis_index("c")
        s = lax.axis_index("s")
        base = (c * NS + s) * per_tile
        gsems = (gsem0, gsem1)

        pltpu.sync_copy(zro_hbm, acc.at[pl.ds(s * stripe, stripe)])
        plsc.subcore_barrier()

        def _fetch(b, j):
            off = base + j * k
            pltpu.sync_copy(qs_hbm.at[pl.ds(off, k)], sidx.at[b])
            pltpu.sync_copy(qd_hbm.at[pl.ds(off, k)], didx.at[b])
            pltpu.async_copy(xwn_hbm.at[sidx.at[b]], rows.at[b], gsems[b])

        _fetch(0, 0)
        _fetch(1, 1)

        def _step(i, carry):
            for b in range(2):
                j = 2 * i + b
                pltpu.make_async_copy(xwn_hbm.at[sidx.at[b]],
                                      rows.at[b], gsems[b]).wait()
                pltpu.sync_copy(rows.at[b], acc.at[didx.at[b]], add=True)

                @pl.when(j + 2 < nchunks)
                def _():
                    _fetch(b, j + 2)
            return carry

        lax.fori_loop(0, nchunks // 2, _step, 0)
        plsc.subcore_barrier()
        pltpu.sync_copy(acc.at[pl.ds(s * stripe, stripe)],
                        out_hbm.at[c, pl.ds(s * stripe, stripe)])

    return kern(xwn, qs, qd, zeros_stripe)


def _k4a_body(x_ref, w1_ref, b1_ref, y1_ref, s1_ref, q1_ref):
    i = pl.program_id(0)

    @pl.when(i == 0)
    def _():
        s1_ref[...] = jnp.zeros_like(s1_ref)
        q1_ref[...] = jnp.zeros_like(q1_ref)

    y = b1_ref[...]
    for h in range(HEADS):
        y = y + jnp.dot(x_ref[h], w1_ref[h],
                        preferred_element_type=jnp.float32)
    y1_ref[...] = y
    s1_ref[...] += jnp.sum(y, axis=0, keepdims=True)
    q1_ref[...] += jnp.sum(y * y, axis=0, keepdims=True)


def _k4b_body(y1_ref, s1_ref, q1_ref, g1_ref, be1_ref, w2_ref, b2_ref,
              y2_ref, s2_ref, q2_ref):
    i = pl.program_id(0)

    @pl.when(i == 0)
    def _():
        s2_ref[...] = jnp.zeros_like(s2_ref)
        q2_ref[...] = jnp.zeros_like(q2_ref)

    n = jnp.float32(N_T)
    mn = s1_ref[...] / n
    vr = q1_ref[...] / n - mn * mn
    a = _elu((y1_ref[...] - mn) / jnp.sqrt(vr + 1e-5) * g1_ref[...] + be1_ref[...])
    y = jnp.dot(a, w2_ref[...], preferred_element_type=jnp.float32) + b2_ref[...]
    y2_ref[...] = y
    s2_ref[...] += jnp.sum(y, axis=0, keepdims=True)
    q2_ref[...] += jnp.sum(y * y, axis=0, keepdims=True)


def _k4c_body(y2_ref, s2_ref, q2_ref, g2_ref, be2_ref, htg_ref, out_ref):
    n = jnp.float32(N_T)
    mn = s2_ref[...] / n
    vr = q2_ref[...] / n - mn * mn
    mlp = _elu((y2_ref[...] - mn) / jnp.sqrt(vr + 1e-5) * g2_ref[...] + be2_ref[...])
    htg = htg_ref[...]
    out_ref[...] = jnp.where(htg != 0, mlp + htg, htg)


def _mlp1_residual(x, htg, p):
    m = p['mlp1']
    rb = 1000
    grid = (N_T // rb,)
    dh = m['W1'].shape[1]
    arb = pltpu.CompilerParams(dimension_semantics=("arbitrary",))
    y1, s1, q1 = pl.pallas_call(
        _k4a_body,
        grid=grid,
        in_specs=[
            pl.BlockSpec((HEADS, rb, H), lambda i: (0, i, 0)),
            pl.BlockSpec((HEADS, H, dh), lambda i: (0, 0, 0)),
            pl.BlockSpec((1, dh), lambda i: (0, 0)),
        ],
        out_specs=[
            pl.BlockSpec((rb, dh), lambda i: (i, 0)),
            pl.BlockSpec((1, dh), lambda i: (0, 0)),
            pl.BlockSpec((1, dh), lambda i: (0, 0)),
        ],
        out_shape=[
            jax.ShapeDtypeStruct((N_T, dh), jnp.float32),
            jax.ShapeDtypeStruct((1, dh), jnp.float32),
            jax.ShapeDtypeStruct((1, dh), jnp.float32),
        ],
        compiler_params=arb,
    )(x, m['W1'].reshape(HEADS, H, dh), m['b1'].reshape(1, -1))
    y2, s2, q2 = pl.pallas_call(
        _k4b_body,
        grid=grid,
        in_specs=[
            pl.BlockSpec((rb, dh), lambda i: (i, 0)),
            pl.BlockSpec((1, dh), lambda i: (0, 0)),
            pl.BlockSpec((1, dh), lambda i: (0, 0)),
            pl.BlockSpec((1, dh), lambda i: (0, 0)),
            pl.BlockSpec((1, dh), lambda i: (0, 0)),
            pl.BlockSpec((dh, H), lambda i: (0, 0)),
            pl.BlockSpec((1, H), lambda i: (0, 0)),
        ],
        out_specs=[
            pl.BlockSpec((rb, H), lambda i: (i, 0)),
            pl.BlockSpec((1, H), lambda i: (0, 0)),
            pl.BlockSpec((1, H), lambda i: (0, 0)),
        ],
        out_shape=[
            jax.ShapeDtypeStruct((N_T, H), jnp.float32),
            jax.ShapeDtypeStruct((1, H), jnp.float32),
            jax.ShapeDtypeStruct((1, H), jnp.float32),
        ],
        compiler_params=arb,
    )(y1, s1, q1, m['g1'].reshape(1, -1), m['be1'].reshape(1, -1),
      m['W2'], m['b2'].reshape(1, -1))
    return pl.pallas_call(
        _k4c_body,
        grid=grid,
        in_specs=[
            pl.BlockSpec((rb, H), lambda i: (i, 0)),
            pl.BlockSpec((1, H), lambda i: (0, 0)),
            pl.BlockSpec((1, H), lambda i: (0, 0)),
            pl.BlockSpec((1, H), lambda i: (0, 0)),
            pl.BlockSpec((1, H), lambda i: (0, 0)),
            pl.BlockSpec((rb, H), lambda i: (i, 0)),
        ],
        out_specs=pl.BlockSpec((rb, H), lambda i: (i, 0)),
        out_shape=jax.ShapeDtypeStruct((N_T, H), jnp.float32),
    )(y2, s2, q2, m['g2'].reshape(1, -1), m['be2'].reshape(1, -1), htg)


def _kxwn_body(n_ref, w_ref, deg_ref, xwn_ref, dinv_ref):
    xw = jnp.dot(n_ref[...], w_ref[...],
                 preferred_element_type=jnp.float32)
    deg = deg_ref[0][:, 0:1] + deg_ref[1][:, 0:1] + 1.0
    dinv_col = deg ** -0.5
    dinv_ref[...] = dinv_col
    xwn_ref[...] = xw * dinv_col


def _xwn(n, gcn_W, deg_t):
    full = lambda shp: pl.BlockSpec(shp, lambda: tuple(0 for _ in shp))
    return pl.pallas_call(
        _kxwn_body,
        in_specs=[full(n.shape), full(gcn_W.shape), full(deg_t.shape)],
        out_specs=[full((N_Q, H)), full((N_Q, 1))],
        out_shape=[
            jax.ShapeDtypeStruct((N_Q, H), jnp.float32),
            jax.ShapeDtypeStruct((N_Q, 1), jnp.float32),
        ],
    )(n, gcn_W, deg_t)


def _k6_body(p0_ref, p1_ref, xwn_ref, dinv_ref, bias_ref, hq_ref,
             w1_ref, b1_ref, g1_ref, be1_ref,
             w2_ref, b2_ref, g2_ref, be2_ref, out_ref):
    x = (dinv_ref[...] * (p0_ref[...] + p1_ref[...] + xwn_ref[...])
         + bias_ref[...])
    y = jnp.dot(x, w1_ref[...], preferred_element_type=jnp.float32) + b1_ref[...]
    mn = jnp.mean(y, axis=0, keepdims=True)
    vr = jnp.mean((y - mn) ** 2, axis=0, keepdims=True)
    y = _elu((y - mn) / jnp.sqrt(vr + 1e-5) * g1_ref[...] + be1_ref[...])
    y = jnp.dot(y, w2_ref[...], preferred_element_type=jnp.float32) + b2_ref[...]
    mn2 = jnp.mean(y, axis=0, keepdims=True)
    vr2 = jnp.mean((y - mn2) ** 2, axis=0, keepdims=True)
    y = _elu((y - mn2) / jnp.sqrt(vr2 + 1e-5) * g2_ref[...] + be2_ref[...])
    out_ref[...] = y + hq_ref[...]


def _mlp2_residual(p0, p1, xwn, dinv_col, bias, h_q, p):
    m = p['mlp2']
    full = lambda shp: pl.BlockSpec(shp, lambda: (0, 0))
    args = [p0, p1, xwn, dinv_col, bias, h_q,
            m['W1'], m['b1'].reshape(1, -1), m['g1'].reshape(1, -1),
            m['be1'].reshape(1, -1),
            m['W2'], m['b2'].reshape(1, -1), m['g2'].reshape(1, -1),
            m['be2'].reshape(1, -1)]
    return pl.pallas_call(
        _k6_body,
        in_specs=[full(a.shape) for a in args],
        out_specs=full((N_Q, H)),
        out_shape=jax.ShapeDtypeStruct((N_Q, H), jnp.float32),
    )(*args)


def _k5_body(hq_ref, ht_ref, mask_ref, stau_ref, mm_ref, mmat_ref):
    hq = hq_ref[...]
    ht = ht_ref[...]
    dot = jax.lax.dot_general(hq, ht, (((1,), (1,)), ((), ())),
                              preferred_element_type=jnp.float32)
    nq = jnp.sqrt(jnp.sum(hq * hq, axis=1, keepdims=True))
    ones = jnp.ones((1, H), jnp.float32)
    nt2 = jax.lax.dot_general(ones, ht * ht, (((1,), (1,)), ((), ())),
                              preferred_element_type=jnp.float32)
    nrm = nq * jnp.sqrt(nt2)
    cosm = jnp.where(nrm != 0, dot / (nrm + 1e-9), -1.0)
    stau = stau_ref[0, 0]
    maskf = mask_ref[...].astype(jnp.float32)
    mmat = cosm * maskf / stau + (-1e9) * (1.0 - maskf)
    mx = jnp.max(mmat, axis=1, keepdims=True)
    e = jnp.exp(mmat - mx)
    s = jnp.sum(e, axis=1, keepdims=True)
    mm_ref[...] = e / s
    mmat_ref[...] = mmat * stau


def _matching(h_q_new, h_t_new, mask_i8, stau):
    rb = 128
    grid = (N_Q // rb,)
    return pl.pallas_call(
        _k5_body,
        grid=grid,
        in_specs=[
            pl.BlockSpec((rb, H), lambda i: (i, 0)),
            pl.BlockSpec((N_T, H), lambda i: (0, 0)),
            pl.BlockSpec((rb, N_T), lambda i: (i, 0)),
            pl.BlockSpec((1, 1), lambda i: (0, 0)),
        ],
        out_specs=[
            pl.BlockSpec((rb, N_T), lambda i: (i, 0)),
            pl.BlockSpec((rb, N_T), lambda i: (i, 0)),
        ],
        out_shape=[
            jax.ShapeDtypeStruct((N_Q, N_T), jnp.float32),
            jax.ShapeDtypeStruct((N_Q, N_T), jnp.float32),
        ],
    )(h_q_new, h_t_new, mask_i8, stau)


def kernel(h_t, h_q, mm, matching_matrix, mask, target_edge_index,
           target_batch, query_edge_index, query_batch, params, ly):
    p = params
    thr = jnp.tanh(p['beta']).reshape(1, 1)
    stau = jax.nn.sigmoid(p['tau']).reshape(1, 1)

    htg = _gate_ht(matching_matrix, h_t, thr)
    n = _matmul(mm, htg, 256, N_T)

    q_mlp = _pool_mlp0(h_q, query_batch.reshape(N_Q, 1).astype(jnp.int32), p)
    a0 = q_mlp.reshape(NB, HEADS, 2 * H)[:, :, :H].reshape(NB, HEADS * H)

    z3 = _gat_messages(htg, target_batch.reshape(N_T, 1).astype(jnp.int32),
                       a0, p['gat_W'])

    src = target_edge_index[0].astype(jnp.int32)
    dst = target_edge_index[1].astype(jnp.int32)
    h_t_gat = _gat_sc(z3, src, dst,
                      jnp.zeros((_STRIPE_T, H), jnp.float32))

    qs = query_edge_index[0].astype(jnp.int32)
    qd = query_edge_index[1].astype(jnp.int32)
    zq_stripe = jnp.zeros((N_Q // NS, H), jnp.float32)
    deg_t = _gcn_sc(jnp.ones((N_Q, H), jnp.float32), qs, qd, zq_stripe)
    xwn, dinv_col = _xwn(n, p['gcn_W'], deg_t)
    part = _gcn_sc(xwn, qs, qd, zq_stripe)

    h_t_new = _mlp1_residual(h_t_gat, htg, p)
    h_q_new = _mlp2_residual(part[0], part[1], xwn, dinv_col,
                             jnp.broadcast_to(p['gcn_b'].reshape(1, H), (N_Q, H)),
                             h_q, p)

    mm_out, mmat_stau = _matching(h_q_new, h_t_new,
                                  mask.astype(jnp.int8), stau)
    return (h_t_new, h_q_new, mm_out, mmat_stau)
